# baseline (speedup 1.0000x reference)
import jax
import jax.numpy as jnp
from jax import lax
from jax.experimental import pallas as pl
from jax.experimental.pallas import tpu as pltpu
from jax.experimental.pallas import tpu_sc as plsc

T = 2048
H = 1024
F = 1024
E = 8
TOP_K = 2
N_GROUP = 4
SF = 2048

A = T * TOP_K
NTILES = 32
CHUNK = A // NTILES
TPW = T // NTILES
BR = 256
NB = A // BR + E
PAD_ROWS = NB * BR
BT = 512


def _router_body(x_ref, gw_ref, tw_ref, te_ref, start_ref, bnd_ref):
    x32 = x_ref[...]
    logits = jnp.dot(x32, gw_ref[...].T, preferred_element_type=jnp.float32)
    s = jax.nn.softmax(logits, axis=-1)
    g = jnp.max(s.reshape(T, N_GROUP, E // N_GROUP), axis=-1)
    jidx = lax.broadcasted_iota(jnp.int32, (T, N_GROUP), 1)
    m1 = jnp.max(g, axis=-1, keepdims=True)
    i1 = jnp.min(jnp.where(g == m1, jidx, N_GROUP), axis=-1, keepdims=True)
    g2 = jnp.where(jidx == i1, -1.0, g)
    m2 = jnp.max(g2, axis=-1, keepdims=True)
    i2 = jnp.min(jnp.where(g2 == m2, jidx, N_GROUP), axis=-1, keepdims=True)
    eidx = lax.broadcasted_iota(jnp.int32, (T, E), 1)
    gid = eidx // (E // N_GROUP)
    keep = (gid == i1) | (gid == i2)
    sm = jnp.where(keep, s, 0.0)
    w1 = jnp.max(sm, axis=-1, keepdims=True)
    e1 = jnp.min(jnp.where(sm == w1, eidx, E), axis=-1, keepdims=True)
    sm2 = jnp.where(eidx == e1, -1.0, sm)
    w2v = jnp.max(sm2, axis=-1, keepdims=True)
    e2 = jnp.min(jnp.where(sm2 == w2v, eidx, E), axis=-1, keepdims=True)
    denom = w1 + w2v + 1e-20
    tw_ref[...] = jnp.concatenate([w1 / denom, w2v / denom], axis=1)
    te_ref[...] = jnp.concatenate([e1, e2], axis=1)

    onehot = ((eidx == e1).astype(jnp.float32)
              + (eidx == e2).astype(jnp.float32))
    r32 = lax.broadcasted_iota(jnp.int32, (NTILES, T), 0)
    c32 = lax.broadcasted_iota(jnp.int32, (NTILES, T), 1)
    sel = (c32 // TPW == r32).astype(jnp.float32)
    hist = jnp.dot(sel, onehot, preferred_element_type=jnp.float32)
    ri = lax.broadcasted_iota(jnp.int32, (NTILES, NTILES), 0)
    ci = lax.broadcasted_iota(jnp.int32, (NTILES, NTILES), 1)
    lt = (ci < ri).astype(jnp.float32)
    excl = jnp.dot(lt, hist, preferred_element_type=jnp.float32)
    counts = jnp.sum(hist, axis=0, keepdims=True)
    padded = jnp.floor((counts + (BR - 1)) / BR) * BR
    r8 = lax.broadcasted_iota(jnp.int32, (E, E), 0)
    c8 = lax.broadcasted_iota(jnp.int32, (E, E), 1)
    m8 = (r8 < c8).astype(jnp.float32)
    po = jnp.dot(padded, m8, preferred_element_type=jnp.float32)
    bnd = po + padded
    start = (excl + po).astype(jnp.int32)
    start_ref[...] = jnp.concatenate(
        [start, jnp.zeros((NTILES, 8), jnp.int32)], axis=1)
    bnd_ref[...] = jnp.broadcast_to(
        jnp.concatenate([bnd.astype(jnp.int32),
                         jnp.full((1, 8), PAD_ROWS, jnp.int32)], axis=1),
        (8, 16))


def _run_router(hidden, gate_w):
    return pl.pallas_call(
        _router_body,
        grid=(1,),
        in_specs=[
            pl.BlockSpec((T, H), lambda i: (0, 0)),
            pl.BlockSpec((E, H), lambda i: (0, 0)),
        ],
        out_specs=[
            pl.BlockSpec((T, 2), lambda i: (0, 0)),
            pl.BlockSpec((T, 2), lambda i: (0, 0)),
            pl.BlockSpec((NTILES, 16), lambda i: (0, 0)),
            pl.BlockSpec((8, 16), lambda i: (0, 0)),
        ],
        out_shape=[
            jax.ShapeDtypeStruct((T, 2), jnp.float32),
            jax.ShapeDtypeStruct((T, 2), jnp.int32),
            jax.ShapeDtypeStruct((NTILES, 16), jnp.int32),
            jax.ShapeDtypeStruct((8, 16), jnp.int32),
        ],
    )(hidden, gate_w)


def _dispatch_body(hid_hbm, te_hbm, start_hbm, bnd_hbm,
                   xs_hbm, da_hbm, db_hbm, bm_hbm,
                   hchunk, ev, dest, da, db, startrow, bndv, bmv,
                   sem, sem2):
    c = lax.axis_index("c")
    s = lax.axis_index("s")
    wid = s * 2 + c
    hload = pltpu.async_copy(hid_hbm.at[pl.ds(wid * TPW, TPW), :],
                             hchunk, sem2)
    pltpu.sync_copy(start_hbm.at[pl.ds(wid * 16, 16)], startrow)
    pltpu.sync_copy(te_hbm.at[pl.ds(wid * CHUNK, CHUNK)], ev)
    lane = lax.iota(jnp.int32, 16)
    cnt = startrow[...]
    for j in range(CHUNK // 16):
        evj = ev[pl.ds(j * 16, 16)]
        dst = jnp.zeros((16,), jnp.int32)
        for e in range(E):
            m = evj == e
            m01 = m.astype(jnp.int32)
            incl = plsc.cumsum(m01)
            cnt_e = jnp.sum(jnp.where(lane == e, cnt, 0))
            tot = jnp.sum(m01)
            dst = jnp.where(m, cnt_e + incl - 1, dst)
            cnt = cnt + jnp.where(lane == e, tot, 0)
        dest[pl.ds(j * 16, 16)] = dst
    for j in range(TPW // 16):
        idx_e = j * 32 + 2 * lane
        da[pl.ds(j * 16, 16)] = plsc.load_gather(dest, [idx_e])
        db[pl.ds(j * 16, 16)] = plsc.load_gather(dest, [idx_e + 1])
    hload.wait()
    cp_a = pltpu.async_copy(hchunk, xs_hbm.at[da], sem)
    cp_b = pltpu.async_copy(hchunk, xs_hbm.at[db], sem2)
    pltpu.sync_copy(da, da_hbm.at[pl.ds(wid * TPW, TPW)])
    pltpu.sync_copy(db, db_hbm.at[pl.ds(wid * TPW, TPW)])
    cp_a.wait()
    cp_b.wait()

    @pl.when(wid == 0)
    def _block_map():
        pltpu.sync_copy(bnd_hbm.at[pl.ds(0, 16)], bndv)
        bvals = bndv[...]
        for v in range(4):
            bvec = (lax.iota(jnp.int32, 16) + v * 16) * BR
            acc = jnp.zeros((16,), jnp.int32)
            for e in range(E):
                bnd_e = jnp.sum(jnp.where(lane == e, bvals, 0))
                acc = acc + (bvec >= bnd_e).astype(jnp.int32)
            bmv[pl.ds(v * 16, 16)] = acc
        pltpu.sync_copy(bmv, bm_hbm)


def _run_dispatch(hidden_bf, te_flat, start_flat, bnd_flat):
    mesh = plsc.VectorSubcoreMesh(core_axis_name="c", subcore_axis_name="s")
    return pl.kernel(
        _dispatch_body,
        out_type=[
            jax.ShapeDtypeStruct((PAD_ROWS, H), jnp.float32),
            jax.ShapeDtypeStruct((T,), jnp.int32),
            jax.ShapeDtypeStruct((T,), jnp.int32),
            jax.ShapeDtypeStruct((64,), jnp.int32),
        ],
        mesh=mesh,
        compiler_params=pltpu.CompilerParams(needs_layout_passes=False),
        scratch_types=[
            pltpu.VMEM((TPW, H), jnp.float32),
            pltpu.VMEM((CHUNK,), jnp.int32),
            pltpu.VMEM((CHUNK,), jnp.int32),
            pltpu.VMEM((TPW,), jnp.int32),
            pltpu.VMEM((TPW,), jnp.int32),
            pltpu.VMEM((16,), jnp.int32),
            pltpu.VMEM((16,), jnp.int32),
            pltpu.VMEM((64,), jnp.int32),
            pltpu.SemaphoreType.DMA,
            pltpu.SemaphoreType.DMA,
        ],
    )(hidden_bf, te_flat, start_flat, bnd_flat)


FK = 4
FC = F // FK


def _gemm_body(bm_ref, x_ref, w13g_ref, w13u_ref, w2_ref, out_ref):
    b = pl.program_id(0)
    k = pl.program_id(1)

    @pl.when(bm_ref[b] < E)
    def _compute():
        xb = x_ref[...].astype(jnp.bfloat16)
        g = jnp.dot(xb, w13g_ref[0, 0, 0].T, preferred_element_type=jnp.float32)
        u = jnp.dot(xb, w13u_ref[0, 0, 0].T, preferred_element_type=jnp.float32)
        h2 = (jax.nn.sigmoid(g) * g * u).astype(jnp.bfloat16)
        o = jnp.dot(h2, w2_ref[0, 0].T, preferred_element_type=jnp.float32)

        @pl.when(k == 0)
        def _init():
            out_ref[...] = o

        @pl.when(k > 0)
        def _acc():
            out_ref[...] += o


def _run_gemm(bm, xs, w13_b, w2_b):
    w13_r = w13_b.reshape(E, 2, FK, FC, H)
    w2_r = w2_b.reshape(E, H, FK, FC).swapaxes(1, 2)
    return pl.pallas_call(
        _gemm_body,
        grid_spec=pltpu.PrefetchScalarGridSpec(
            num_scalar_prefetch=1,
            grid=(NB, FK),
            in_specs=[
                pl.BlockSpec((BR, H), lambda b, k, bm: (b, 0)),
                pl.BlockSpec(
                    (1, 1, 1, FC, H),
                    lambda b, k, bm: (jnp.minimum(bm[b], E - 1), 0, k, 0, 0)),
                pl.BlockSpec(
                    (1, 1, 1, FC, H),
                    lambda b, k, bm: (jnp.minimum(bm[b], E - 1), 1, k, 0, 0)),
                pl.BlockSpec(
                    (1, 1, H, FC),
                    lambda b, k, bm: (jnp.minimum(bm[b], E - 1), k, 0, 0)),
            ],
            out_specs=pl.BlockSpec((BR, H), lambda b, k, bm: (b, 0)),
        ),
        out_shape=jax.ShapeDtypeStruct((PAD_ROWS, H), jnp.float32),
        compiler_params=pltpu.CompilerParams(
            dimension_semantics=("arbitrary", "arbitrary"),
        ),
    )(bm, xs, w13_r, w13_r, w2_r)


def _combine_body(os_hbm, da_hbm, db_hbm, ra_hbm, rb_hbm,
                  ia, ib, rows_a, rows_b, sem_a, sem_b):
    c = lax.axis_index("c")
    s = lax.axis_index("s")
    wid = s * 2 + c
    for r in range(2):
        base = wid * TPW + r * 32
        pltpu.sync_copy(da_hbm.at[pl.ds(base, 32)], ia)
        pltpu.sync_copy(db_hbm.at[pl.ds(base, 32)], ib)
        cp_a = pltpu.async_copy(os_hbm.at[ia], rows_a, sem_a)
        cp_b = pltpu.async_copy(os_hbm.at[ib], rows_b, sem_b)
        cp_a.wait()
        pltpu.sync_copy(rows_a, ra_hbm.at[pl.ds(base, 32), :])
        cp_b.wait()
        pltpu.sync_copy(rows_b, rb_hbm.at[pl.ds(base, 32), :])


def _run_combine(os, da, db):
    mesh = plsc.VectorSubcoreMesh(core_axis_name="c", subcore_axis_name="s")
    return pl.kernel(
        _combine_body,
        out_type=[
            jax.ShapeDtypeStruct((T, H), jnp.float32),
            jax.ShapeDtypeStruct((T, H), jnp.float32),
        ],
        mesh=mesh,
        compiler_params=pltpu.CompilerParams(needs_layout_passes=False),
        scratch_types=[
            pltpu.VMEM((32,), jnp.int32),
            pltpu.VMEM((32,), jnp.int32),
            pltpu.VMEM((32, H), jnp.float32),
            pltpu.VMEM((32, H), jnp.float32),
            pltpu.SemaphoreType.DMA,
            pltpu.SemaphoreType.DMA,
        ],
    )(os, da, db)


def _shared_body(x_ref, sw13_ref, sw2_ref, out_ref):
    xb = x_ref[...].astype(jnp.bfloat16)
    h1 = jnp.dot(xb, sw13_ref[...].T, preferred_element_type=jnp.float32)
    g, u = jnp.split(h1, 2, axis=-1)
    h2 = (jax.nn.sigmoid(g) * g * u).astype(jnp.bfloat16)
    out_ref[...] = jnp.dot(h2, sw2_ref[...].T,
                           preferred_element_type=jnp.float32)


def _run_shared(hidden, sw13_b, sw2_b):
    return pl.pallas_call(
        _shared_body,
        grid=(T // BT,),
        in_specs=[
            pl.BlockSpec((BT, H), lambda t: (t, 0)),
            pl.BlockSpec((2 * SF, H), lambda t: (0, 0)),
            pl.BlockSpec((H, SF), lambda t: (0, 0)),
        ],
        out_specs=pl.BlockSpec((BT, H), lambda t: (t, 0)),
        out_shape=jax.ShapeDtypeStruct((T, H), jnp.float32),
        compiler_params=pltpu.CompilerParams(
            dimension_semantics=("parallel",),
        ),
    )(hidden, sw13_b, sw2_b)


def _final_body(sh_ref, ra_ref, rb_ref, tw_ref, out_ref):
    tw = tw_ref[...]
    out_ref[...] = (sh_ref[...]
                    + tw[:, 0:1] * ra_ref[...]
                    + tw[:, 1:2] * rb_ref[...])


def _run_final(shared, ra, rb, tw):
    return pl.pallas_call(
        _final_body,
        grid=(T // BT,),
        in_specs=[
            pl.BlockSpec((BT, H), lambda t: (t, 0)),
            pl.BlockSpec((BT, H), lambda t: (t, 0)),
            pl.BlockSpec((BT, H), lambda t: (t, 0)),
            pl.BlockSpec((BT, 2), lambda t: (t, 0)),
        ],
        out_specs=pl.BlockSpec((BT, H), lambda t: (t, 0)),
        out_shape=jax.ShapeDtypeStruct((T, H), jnp.float32),
        compiler_params=pltpu.CompilerParams(
            dimension_semantics=("parallel",),
        ),
    )(shared, ra, rb, tw)


def kernel(hidden_states, gate_w, w13, w2, shared_w13, shared_w2):
    w13_b = w13.astype(jnp.bfloat16)
    w2_b = w2.astype(jnp.bfloat16)
    sw13_b = shared_w13.astype(jnp.bfloat16)
    sw2_b = shared_w2.astype(jnp.bfloat16)

    tw, te, start, bnd = _run_router(hidden_states, gate_w)
    xs, da, db, bm = _run_dispatch(
        hidden_states, te.reshape(A), start.reshape(NTILES * 16), bnd[0])
    shared = _run_shared(hidden_states, sw13_b, sw2_b)
    os = _run_gemm(bm, xs, w13_b, w2_b)
    ra, rb = _run_combine(os, da, db)
    return _run_final(shared, ra, rb, tw)


# speedup vs baseline: 1.3237x; 1.3237x over previous
import jax
import jax.numpy as jnp
from jax import lax
from jax.experimental import pallas as pl
from jax.experimental.pallas import tpu as pltpu
from jax.experimental.pallas import tpu_sc as plsc

T = 2048
H = 1024
F = 1024
E = 8
TOP_K = 2
N_GROUP = 4
SF = 2048

A = T * TOP_K
NTILES = 32
CHUNK = A // NTILES
TPW = T // NTILES
BR = 512
NB = A // BR + E
PAD_ROWS = NB * BR
BT = 512


def _router_body(x_ref, gw_ref, tw_ref, te_ref, start_ref, bnd_ref):
    x32 = x_ref[...]
    logits = jnp.dot(x32, gw_ref[...].T, preferred_element_type=jnp.float32)
    s = jax.nn.softmax(logits, axis=-1)
    g = jnp.max(s.reshape(T, N_GROUP, E // N_GROUP), axis=-1)
    jidx = lax.broadcasted_iota(jnp.int32, (T, N_GROUP), 1)
    m1 = jnp.max(g, axis=-1, keepdims=True)
    i1 = jnp.min(jnp.where(g == m1, jidx, N_GROUP), axis=-1, keepdims=True)
    g2 = jnp.where(jidx == i1, -1.0, g)
    m2 = jnp.max(g2, axis=-1, keepdims=True)
    i2 = jnp.min(jnp.where(g2 == m2, jidx, N_GROUP), axis=-1, keepdims=True)
    eidx = lax.broadcasted_iota(jnp.int32, (T, E), 1)
    gid = eidx // (E // N_GROUP)
    keep = (gid == i1) | (gid == i2)
    sm = jnp.where(keep, s, 0.0)
    w1 = jnp.max(sm, axis=-1, keepdims=True)
    e1 = jnp.min(jnp.where(sm == w1, eidx, E), axis=-1, keepdims=True)
    sm2 = jnp.where(eidx == e1, -1.0, sm)
    w2v = jnp.max(sm2, axis=-1, keepdims=True)
    e2 = jnp.min(jnp.where(sm2 == w2v, eidx, E), axis=-1, keepdims=True)
    denom = w1 + w2v + 1e-20
    tw_ref[...] = jnp.concatenate([w1 / denom, w2v / denom], axis=1)
    te_ref[...] = jnp.concatenate([e1, e2], axis=1)

    onehot = ((eidx == e1).astype(jnp.float32)
              + (eidx == e2).astype(jnp.float32))
    r32 = lax.broadcasted_iota(jnp.int32, (NTILES, T), 0)
    c32 = lax.broadcasted_iota(jnp.int32, (NTILES, T), 1)
    sel = (c32 // TPW == r32).astype(jnp.float32)
    hist = jnp.dot(sel, onehot, preferred_element_type=jnp.float32)
    ri = lax.broadcasted_iota(jnp.int32, (NTILES, NTILES), 0)
    ci = lax.broadcasted_iota(jnp.int32, (NTILES, NTILES), 1)
    lt = (ci < ri).astype(jnp.float32)
    excl = jnp.dot(lt, hist, preferred_element_type=jnp.float32)
    counts = jnp.sum(hist, axis=0, keepdims=True)
    padded = jnp.floor((counts + (BR - 1)) / BR) * BR
    r8 = lax.broadcasted_iota(jnp.int32, (E, E), 0)
    c8 = lax.broadcasted_iota(jnp.int32, (E, E), 1)
    m8 = (r8 < c8).astype(jnp.float32)
    po = jnp.dot(padded, m8, preferred_element_type=jnp.float32)
    bnd = po + padded
    start = (excl + po).astype(jnp.int32)
    start_ref[...] = jnp.concatenate(
        [start, jnp.zeros((NTILES, 8), jnp.int32)], axis=1)
    bnd_ref[...] = jnp.broadcast_to(
        jnp.concatenate([bnd.astype(jnp.int32),
                         jnp.full((1, 8), PAD_ROWS, jnp.int32)], axis=1),
        (8, 16))


def _run_router(hidden, gate_w):
    return pl.pallas_call(
        _router_body,
        grid=(1,),
        in_specs=[
            pl.BlockSpec((T, H), lambda i: (0, 0)),
            pl.BlockSpec((E, H), lambda i: (0, 0)),
        ],
        out_specs=[
            pl.BlockSpec((T, 2), lambda i: (0, 0)),
            pl.BlockSpec((T, 2), lambda i: (0, 0)),
            pl.BlockSpec((NTILES, 16), lambda i: (0, 0)),
            pl.BlockSpec((8, 16), lambda i: (0, 0)),
        ],
        out_shape=[
            jax.ShapeDtypeStruct((T, 2), jnp.float32),
            jax.ShapeDtypeStruct((T, 2), jnp.int32),
            jax.ShapeDtypeStruct((NTILES, 16), jnp.int32),
            jax.ShapeDtypeStruct((8, 16), jnp.int32),
        ],
    )(hidden, gate_w)


def _dispatch_body(hid_hbm, te_hbm, start_hbm, bnd_hbm,
                   xs_hbm, da_hbm, db_hbm, bm_hbm,
                   hchunk, ev, dest, da, db, startrow, bndv, bmv,
                   sem, sem2):
    c = lax.axis_index("c")
    s = lax.axis_index("s")
    wid = s * 2 + c
    hload = pltpu.async_copy(hid_hbm.at[pl.ds(wid * TPW, TPW), :],
                             hchunk, sem2)
    pltpu.sync_copy(start_hbm.at[pl.ds(wid * 16, 16)], startrow)
    pltpu.sync_copy(te_hbm.at[pl.ds(wid * CHUNK, CHUNK)], ev)
    lane = lax.iota(jnp.int32, 16)
    cnt = startrow[...]
    for j in range(CHUNK // 16):
        evj = ev[pl.ds(j * 16, 16)]
        dst = jnp.zeros((16,), jnp.int32)
        for e in range(E):
            m = evj == e
            m01 = m.astype(jnp.int32)
            incl = plsc.cumsum(m01)
            cnt_e = jnp.sum(jnp.where(lane == e, cnt, 0))
            tot = jnp.sum(m01)
            dst = jnp.where(m, cnt_e + incl - 1, dst)
            cnt = cnt + jnp.where(lane == e, tot, 0)
        dest[pl.ds(j * 16, 16)] = dst
    for j in range(TPW // 16):
        idx_e = j * 32 + 2 * lane
        da[pl.ds(j * 16, 16)] = plsc.load_gather(dest, [idx_e])
        db[pl.ds(j * 16, 16)] = plsc.load_gather(dest, [idx_e + 1])
    hload.wait()
    cp_a = pltpu.async_copy(hchunk, xs_hbm.at[da], sem)
    cp_b = pltpu.async_copy(hchunk, xs_hbm.at[db], sem2)
    pltpu.sync_copy(da, da_hbm.at[pl.ds(wid * TPW, TPW)])
    pltpu.sync_copy(db, db_hbm.at[pl.ds(wid * TPW, TPW)])
    cp_a.wait()
    cp_b.wait()

    @pl.when(wid == 0)
    def _block_map():
        pltpu.sync_copy(bnd_hbm.at[pl.ds(0, 16)], bndv)
        bvals = bndv[...]
        for v in range(4):
            bvec = (lax.iota(jnp.int32, 16) + v * 16) * BR
            acc = jnp.zeros((16,), jnp.int32)
            for e in range(E):
                bnd_e = jnp.sum(jnp.where(lane == e, bvals, 0))
                acc = acc + (bvec >= bnd_e).astype(jnp.int32)
            bmv[pl.ds(v * 16, 16)] = acc
        pltpu.sync_copy(bmv, bm_hbm)


def _run_dispatch(hidden_bf, te_flat, start_flat, bnd_flat):
    mesh = plsc.VectorSubcoreMesh(core_axis_name="c", subcore_axis_name="s")
    return pl.kernel(
        _dispatch_body,
        out_type=[
            jax.ShapeDtypeStruct((PAD_ROWS, H), jnp.float32),
            jax.ShapeDtypeStruct((T,), jnp.int32),
            jax.ShapeDtypeStruct((T,), jnp.int32),
            jax.ShapeDtypeStruct((64,), jnp.int32),
        ],
        mesh=mesh,
        compiler_params=pltpu.CompilerParams(needs_layout_passes=False),
        scratch_types=[
            pltpu.VMEM((TPW, H), jnp.float32),
            pltpu.VMEM((CHUNK,), jnp.int32),
            pltpu.VMEM((CHUNK,), jnp.int32),
            pltpu.VMEM((TPW,), jnp.int32),
            pltpu.VMEM((TPW,), jnp.int32),
            pltpu.VMEM((16,), jnp.int32),
            pltpu.VMEM((16,), jnp.int32),
            pltpu.VMEM((64,), jnp.int32),
            pltpu.SemaphoreType.DMA,
            pltpu.SemaphoreType.DMA,
        ],
    )(hidden_bf, te_flat, start_flat, bnd_flat)


def _gemm_body(bm_ref, x_ref, w13_ref, w2_ref, out_ref):
    b = pl.program_id(0)

    @pl.when(bm_ref[b] < E)
    def _compute():
        xb = x_ref[...].astype(jnp.bfloat16)
        h1 = jnp.dot(xb, w13_ref[0].T, preferred_element_type=jnp.float32)
        g, u = jnp.split(h1, 2, axis=-1)
        h2 = (jax.nn.sigmoid(g) * g * u).astype(jnp.bfloat16)
        out_ref[...] = jnp.dot(
            h2, w2_ref[0].T, preferred_element_type=jnp.float32)


def _run_gemm(bm, xs, w13_b, w2_b):
    return pl.pallas_call(
        _gemm_body,
        grid_spec=pltpu.PrefetchScalarGridSpec(
            num_scalar_prefetch=1,
            grid=(NB,),
            in_specs=[
                pl.BlockSpec((BR, H), lambda b, bm: (b, 0)),
                pl.BlockSpec((1, 2 * F, H),
                             lambda b, bm: (jnp.minimum(bm[b], E - 1), 0, 0)),
                pl.BlockSpec((1, H, F),
                             lambda b, bm: (jnp.minimum(bm[b], E - 1), 0, 0)),
            ],
            out_specs=pl.BlockSpec((BR, H), lambda b, bm: (b, 0)),
        ),
        out_shape=jax.ShapeDtypeStruct((PAD_ROWS, H), jnp.float32),
        compiler_params=pltpu.CompilerParams(
            dimension_semantics=("arbitrary",),
        ),
    )(bm, xs, w13_b, w2_b)


def _combine_body(os_hbm, da_hbm, db_hbm, ra_hbm, rb_hbm,
                  ia, ib, rows_a, rows_b, sem_a, sem_b):
    c = lax.axis_index("c")
    s = lax.axis_index("s")
    wid = s * 2 + c
    for r in range(2):
        base = wid * TPW + r * 32
        pltpu.sync_copy(da_hbm.at[pl.ds(base, 32)], ia)
        pltpu.sync_copy(db_hbm.at[pl.ds(base, 32)], ib)
        cp_a = pltpu.async_copy(os_hbm.at[ia], rows_a, sem_a)
        cp_b = pltpu.async_copy(os_hbm.at[ib], rows_b, sem_b)
        cp_a.wait()
        pltpu.sync_copy(rows_a, ra_hbm.at[pl.ds(base, 32), :])
        cp_b.wait()
        pltpu.sync_copy(rows_b, rb_hbm.at[pl.ds(base, 32), :])


def _run_combine(os, da, db):
    mesh = plsc.VectorSubcoreMesh(core_axis_name="c", subcore_axis_name="s")
    return pl.kernel(
        _combine_body,
        out_type=[
            jax.ShapeDtypeStruct((T, H), jnp.float32),
            jax.ShapeDtypeStruct((T, H), jnp.float32),
        ],
        mesh=mesh,
        compiler_params=pltpu.CompilerParams(needs_layout_passes=False),
        scratch_types=[
            pltpu.VMEM((32,), jnp.int32),
            pltpu.VMEM((32,), jnp.int32),
            pltpu.VMEM((32, H), jnp.float32),
            pltpu.VMEM((32, H), jnp.float32),
            pltpu.SemaphoreType.DMA,
            pltpu.SemaphoreType.DMA,
        ],
    )(os, da, db)


def _shared_body(x_ref, sw13_ref, sw2_ref, out_ref):
    xb = x_ref[...].astype(jnp.bfloat16)
    h1 = jnp.dot(xb, sw13_ref[...].T, preferred_element_type=jnp.float32)
    g, u = jnp.split(h1, 2, axis=-1)
    h2 = (jax.nn.sigmoid(g) * g * u).astype(jnp.bfloat16)
    out_ref[...] = jnp.dot(h2, sw2_ref[...].T,
                           preferred_element_type=jnp.float32)


def _run_shared(hidden, sw13_b, sw2_b):
    return pl.pallas_call(
        _shared_body,
        grid=(T // BT,),
        in_specs=[
            pl.BlockSpec((BT, H), lambda t: (t, 0)),
            pl.BlockSpec((2 * SF, H), lambda t: (0, 0)),
            pl.BlockSpec((H, SF), lambda t: (0, 0)),
        ],
        out_specs=pl.BlockSpec((BT, H), lambda t: (t, 0)),
        out_shape=jax.ShapeDtypeStruct((T, H), jnp.float32),
        compiler_params=pltpu.CompilerParams(
            dimension_semantics=("parallel",),
        ),
    )(hidden, sw13_b, sw2_b)


def _final_body(sh_ref, ra_ref, rb_ref, tw_ref, out_ref):
    tw = tw_ref[...]
    out_ref[...] = (sh_ref[...]
                    + tw[:, 0:1] * ra_ref[...]
                    + tw[:, 1:2] * rb_ref[...])


def _run_final(shared, ra, rb, tw):
    return pl.pallas_call(
        _final_body,
        grid=(T // BT,),
        in_specs=[
            pl.BlockSpec((BT, H), lambda t: (t, 0)),
            pl.BlockSpec((BT, H), lambda t: (t, 0)),
            pl.BlockSpec((BT, H), lambda t: (t, 0)),
            pl.BlockSpec((BT, 2), lambda t: (t, 0)),
        ],
        out_specs=pl.BlockSpec((BT, H), lambda t: (t, 0)),
        out_shape=jax.ShapeDtypeStruct((T, H), jnp.float32),
        compiler_params=pltpu.CompilerParams(
            dimension_semantics=("parallel",),
        ),
    )(shared, ra, rb, tw)


def kernel(hidden_states, gate_w, w13, w2, shared_w13, shared_w2):
    w13_b = w13.astype(jnp.bfloat16)
    w2_b = w2.astype(jnp.bfloat16)
    sw13_b = shared_w13.astype(jnp.bfloat16)
    sw2_b = shared_w2.astype(jnp.bfloat16)

    tw, te, start, bnd = _run_router(hidden_states, gate_w)
    xs, da, db, bm = _run_dispatch(
        hidden_states, te.reshape(A), start.reshape(NTILES * 16), bnd[0])
    shared = _run_shared(hidden_states, sw13_b, sw2_b)
    os = _run_gemm(bm, xs, w13_b, w2_b)
    ra, rb = _run_combine(os, da, db)
    return _run_final(shared, ra, rb, tw)


# speedup vs baseline: 1.3885x; 1.0490x over previous
import jax
import jax.numpy as jnp
from jax import lax
from jax.experimental import pallas as pl
from jax.experimental.pallas import tpu as pltpu
from jax.experimental.pallas import tpu_sc as plsc

T = 2048
H = 1024
F = 1024
E = 8
TOP_K = 2
N_GROUP = 4
SF = 2048

A = T * TOP_K
NTILES = 32
CHUNK = A // NTILES
TPW = T // NTILES
BR = 512
NB = A // BR + E
PAD_ROWS = NB * BR
BT = 512


def _router_body(x_ref, gw_ref, tw_ref, te_ref, start_ref, bnd_ref):
    x32 = x_ref[...]
    logits = jnp.dot(x32, gw_ref[...].T, preferred_element_type=jnp.float32)
    s = jax.nn.softmax(logits, axis=-1)
    g = jnp.max(s.reshape(T, N_GROUP, E // N_GROUP), axis=-1)
    jidx = lax.broadcasted_iota(jnp.int32, (T, N_GROUP), 1)
    m1 = jnp.max(g, axis=-1, keepdims=True)
    i1 = jnp.min(jnp.where(g == m1, jidx, N_GROUP), axis=-1, keepdims=True)
    g2 = jnp.where(jidx == i1, -1.0, g)
    m2 = jnp.max(g2, axis=-1, keepdims=True)
    i2 = jnp.min(jnp.where(g2 == m2, jidx, N_GROUP), axis=-1, keepdims=True)
    eidx = lax.broadcasted_iota(jnp.int32, (T, E), 1)
    gid = eidx // (E // N_GROUP)
    keep = (gid == i1) | (gid == i2)
    sm = jnp.where(keep, s, 0.0)
    w1 = jnp.max(sm, axis=-1, keepdims=True)
    e1 = jnp.min(jnp.where(sm == w1, eidx, E), axis=-1, keepdims=True)
    sm2 = jnp.where(eidx == e1, -1.0, sm)
    w2v = jnp.max(sm2, axis=-1, keepdims=True)
    e2 = jnp.min(jnp.where(sm2 == w2v, eidx, E), axis=-1, keepdims=True)
    denom = w1 + w2v + 1e-20
    tw_ref[...] = jnp.concatenate([w1 / denom, w2v / denom], axis=1)
    te_ref[...] = jnp.concatenate([e1, e2], axis=1)

    onehot = ((eidx == e1).astype(jnp.float32)
              + (eidx == e2).astype(jnp.float32))
    r32 = lax.broadcasted_iota(jnp.int32, (NTILES, T), 0)
    c32 = lax.broadcasted_iota(jnp.int32, (NTILES, T), 1)
    sel = (c32 // TPW == r32).astype(jnp.float32)
    hist = jnp.dot(sel, onehot, preferred_element_type=jnp.float32)
    ri = lax.broadcasted_iota(jnp.int32, (NTILES, NTILES), 0)
    ci = lax.broadcasted_iota(jnp.int32, (NTILES, NTILES), 1)
    lt = (ci < ri).astype(jnp.float32)
    excl = jnp.dot(lt, hist, preferred_element_type=jnp.float32)
    counts = jnp.sum(hist, axis=0, keepdims=True)
    padded = jnp.floor((counts + (BR - 1)) / BR) * BR
    r8 = lax.broadcasted_iota(jnp.int32, (E, E), 0)
    c8 = lax.broadcasted_iota(jnp.int32, (E, E), 1)
    m8 = (r8 < c8).astype(jnp.float32)
    po = jnp.dot(padded, m8, preferred_element_type=jnp.float32)
    bnd = po + padded
    start = (excl + po).astype(jnp.int32)
    start_ref[...] = jnp.concatenate(
        [start, jnp.zeros((NTILES, 8), jnp.int32)], axis=1)
    bnd_ref[...] = jnp.broadcast_to(
        jnp.concatenate([bnd.astype(jnp.int32),
                         jnp.full((1, 8), PAD_ROWS, jnp.int32)], axis=1),
        (8, 16))


def _run_router(hidden, gate_w):
    return pl.pallas_call(
        _router_body,
        grid=(1,),
        in_specs=[
            pl.BlockSpec((T, H), lambda i: (0, 0)),
            pl.BlockSpec((E, H), lambda i: (0, 0)),
        ],
        out_specs=[
            pl.BlockSpec((T, 2), lambda i: (0, 0)),
            pl.BlockSpec((T, 2), lambda i: (0, 0)),
            pl.BlockSpec((NTILES, 16), lambda i: (0, 0)),
            pl.BlockSpec((8, 16), lambda i: (0, 0)),
        ],
        out_shape=[
            jax.ShapeDtypeStruct((T, 2), jnp.float32),
            jax.ShapeDtypeStruct((T, 2), jnp.int32),
            jax.ShapeDtypeStruct((NTILES, 16), jnp.int32),
            jax.ShapeDtypeStruct((8, 16), jnp.int32),
        ],
    )(hidden, gate_w)


def _dispatch_body(hid_hbm, te_hbm, start_hbm, bnd_hbm,
                   xs_hbm, da_hbm, db_hbm, bm_hbm,
                   hchunk, ev, dest, da, db, startrow, bndv, bmv,
                   sem, sem2):
    c = lax.axis_index("c")
    s = lax.axis_index("s")
    wid = s * 2 + c
    hload = pltpu.async_copy(hid_hbm.at[pl.ds(wid * TPW, TPW), :],
                             hchunk, sem2)
    pltpu.sync_copy(start_hbm.at[pl.ds(wid * 16, 16)], startrow)
    pltpu.sync_copy(te_hbm.at[pl.ds(wid * CHUNK, CHUNK)], ev)
    lane = lax.iota(jnp.int32, 16)
    cnt = startrow[...]
    for j in range(CHUNK // 16):
        evj = ev[pl.ds(j * 16, 16)]
        dst = jnp.zeros((16,), jnp.int32)
        for e in range(E):
            m = evj == e
            m01 = m.astype(jnp.int32)
            incl = plsc.cumsum(m01)
            cnt_e = jnp.sum(jnp.where(lane == e, cnt, 0))
            tot = jnp.sum(m01)
            dst = jnp.where(m, cnt_e + incl - 1, dst)
            cnt = cnt + jnp.where(lane == e, tot, 0)
        dest[pl.ds(j * 16, 16)] = dst
    for j in range(TPW // 16):
        idx_e = j * 32 + 2 * lane
        da[pl.ds(j * 16, 16)] = plsc.load_gather(dest, [idx_e])
        db[pl.ds(j * 16, 16)] = plsc.load_gather(dest, [idx_e + 1])
    hload.wait()
    cp_a = pltpu.async_copy(hchunk, xs_hbm.at[da], sem)
    cp_b = pltpu.async_copy(hchunk, xs_hbm.at[db], sem2)
    pltpu.sync_copy(da, da_hbm.at[pl.ds(wid * TPW, TPW)])
    pltpu.sync_copy(db, db_hbm.at[pl.ds(wid * TPW, TPW)])
    cp_a.wait()
    cp_b.wait()

    @pl.when(wid == 0)
    def _block_map():
        pltpu.sync_copy(bnd_hbm.at[pl.ds(0, 16)], bndv)
        bvals = bndv[...]
        for v in range(4):
            bvec = (lax.iota(jnp.int32, 16) + v * 16) * BR
            acc = jnp.zeros((16,), jnp.int32)
            for e in range(E):
                bnd_e = jnp.sum(jnp.where(lane == e, bvals, 0))
                acc = acc + (bvec >= bnd_e).astype(jnp.int32)
            bmv[pl.ds(v * 16, 16)] = acc
        pltpu.sync_copy(bmv, bm_hbm)


def _run_dispatch(hidden_bf, te_flat, start_flat, bnd_flat):
    mesh = plsc.VectorSubcoreMesh(core_axis_name="c", subcore_axis_name="s")
    return pl.kernel(
        _dispatch_body,
        out_type=[
            jax.ShapeDtypeStruct((PAD_ROWS, H), jnp.float32),
            jax.ShapeDtypeStruct((T,), jnp.int32),
            jax.ShapeDtypeStruct((T,), jnp.int32),
            jax.ShapeDtypeStruct((64,), jnp.int32),
        ],
        mesh=mesh,
        compiler_params=pltpu.CompilerParams(needs_layout_passes=False),
        scratch_types=[
            pltpu.VMEM((TPW, H), jnp.float32),
            pltpu.VMEM((CHUNK,), jnp.int32),
            pltpu.VMEM((CHUNK,), jnp.int32),
            pltpu.VMEM((TPW,), jnp.int32),
            pltpu.VMEM((TPW,), jnp.int32),
            pltpu.VMEM((16,), jnp.int32),
            pltpu.VMEM((16,), jnp.int32),
            pltpu.VMEM((64,), jnp.int32),
            pltpu.SemaphoreType.DMA,
            pltpu.SemaphoreType.DMA,
        ],
    )(hidden_bf, te_flat, start_flat, bnd_flat)


def _gemm_body(bm_ref, x_ref, w13_ref, w2_ref, out_ref):
    b = pl.program_id(0)

    @pl.when(bm_ref[b] < E)
    def _compute():
        xb = x_ref[...].astype(jnp.bfloat16)
        h1 = jnp.dot(xb, w13_ref[0].T, preferred_element_type=jnp.float32)
        g, u = jnp.split(h1, 2, axis=-1)
        h2 = (jax.nn.sigmoid(g) * g * u).astype(jnp.bfloat16)
        out_ref[...] = jnp.dot(
            h2, w2_ref[0].T, preferred_element_type=jnp.float32)


def _run_gemm(bm, xs, w13_b, w2_b):
    return pl.pallas_call(
        _gemm_body,
        grid_spec=pltpu.PrefetchScalarGridSpec(
            num_scalar_prefetch=1,
            grid=(NB,),
            in_specs=[
                pl.BlockSpec((BR, H), lambda b, bm: (b, 0)),
                pl.BlockSpec((1, 2 * F, H),
                             lambda b, bm: (jnp.minimum(bm[b], E - 1), 0, 0)),
                pl.BlockSpec((1, H, F),
                             lambda b, bm: (jnp.minimum(bm[b], E - 1), 0, 0)),
            ],
            out_specs=pl.BlockSpec((BR, H), lambda b, bm: (b, 0)),
        ),
        out_shape=jax.ShapeDtypeStruct((PAD_ROWS, H), jnp.float32),
        compiler_params=pltpu.CompilerParams(
            dimension_semantics=("arbitrary",),
        ),
    )(bm, xs, w13_b, w2_b)


def _combine_body(os_hbm, da_hbm, db_hbm, ra_hbm, rb_hbm,
                  ia, ib, rows_a, rows_b, sem_a, sem_b):
    c = lax.axis_index("c")
    s = lax.axis_index("s")
    wid = s * 2 + c
    for r in range(2):
        base = wid * TPW + r * 32
        pltpu.sync_copy(da_hbm.at[pl.ds(base, 32)], ia)
        pltpu.sync_copy(db_hbm.at[pl.ds(base, 32)], ib)
        cp_a = pltpu.async_copy(os_hbm.at[ia], rows_a, sem_a)
        cp_b = pltpu.async_copy(os_hbm.at[ib], rows_b, sem_b)
        cp_a.wait()
        pltpu.sync_copy(rows_a, ra_hbm.at[pl.ds(base, 32), :])
        cp_b.wait()
        pltpu.sync_copy(rows_b, rb_hbm.at[pl.ds(base, 32), :])


def _run_combine(os, da, db):
    mesh = plsc.VectorSubcoreMesh(core_axis_name="c", subcore_axis_name="s")
    return pl.kernel(
        _combine_body,
        out_type=[
            jax.ShapeDtypeStruct((T, H), jnp.float32),
            jax.ShapeDtypeStruct((T, H), jnp.float32),
        ],
        mesh=mesh,
        compiler_params=pltpu.CompilerParams(needs_layout_passes=False),
        scratch_types=[
            pltpu.VMEM((32,), jnp.int32),
            pltpu.VMEM((32,), jnp.int32),
            pltpu.VMEM((32, H), jnp.float32),
            pltpu.VMEM((32, H), jnp.float32),
            pltpu.SemaphoreType.DMA,
            pltpu.SemaphoreType.DMA,
        ],
    )(os, da, db)


def _final_body(x_ref, sw13_ref, sw2_ref, ra_ref, rb_ref, tw_ref, out_ref):
    xb = x_ref[...].astype(jnp.bfloat16)
    h1 = jnp.dot(xb, sw13_ref[...].T, preferred_element_type=jnp.float32)
    g, u = jnp.split(h1, 2, axis=-1)
    h2 = (jax.nn.sigmoid(g) * g * u).astype(jnp.bfloat16)
    shared = jnp.dot(h2, sw2_ref[...].T, preferred_element_type=jnp.float32)
    tw = tw_ref[...]
    out_ref[...] = (shared + tw[:, 0:1] * ra_ref[...]
                    + tw[:, 1:2] * rb_ref[...])


def _run_final(hidden, sw13_b, sw2_b, ra, rb, tw):
    return pl.pallas_call(
        _final_body,
        grid=(T // BT,),
        in_specs=[
            pl.BlockSpec((BT, H), lambda t: (t, 0)),
            pl.BlockSpec((2 * SF, H), lambda t: (0, 0)),
            pl.BlockSpec((H, SF), lambda t: (0, 0)),
            pl.BlockSpec((BT, H), lambda t: (t, 0)),
            pl.BlockSpec((BT, H), lambda t: (t, 0)),
            pl.BlockSpec((BT, 2), lambda t: (t, 0)),
        ],
        out_specs=pl.BlockSpec((BT, H), lambda t: (t, 0)),
        out_shape=jax.ShapeDtypeStruct((T, H), jnp.float32),
        compiler_params=pltpu.CompilerParams(
            dimension_semantics=("parallel",),
        ),
    )(hidden, sw13_b, sw2_b, ra, rb, tw)


def kernel(hidden_states, gate_w, w13, w2, shared_w13, shared_w2):
    w13_b = w13.astype(jnp.bfloat16)
    w2_b = w2.astype(jnp.bfloat16)
    sw13_b = shared_w13.astype(jnp.bfloat16)
    sw2_b = shared_w2.astype(jnp.bfloat16)

    tw, te, start, bnd = _run_router(hidden_states, gate_w)
    xs, da, db, bm = _run_dispatch(
        hidden_states, te.reshape(A), start.reshape(NTILES * 16), bnd[0])
    os = _run_gemm(bm, xs, w13_b, w2_b)
    ra, rb = _run_combine(os, da, db)
    return _run_final(hidden_states, sw13_b, sw2_b, ra, rb, tw)
